# Initial kernel scaffold; baseline (speedup 1.0000x reference)
import jax
import jax.numpy as jnp
from jax.experimental import pallas as pl


def kernel(input_ids, bbox, word_emb, token_type_emb, pos_emb, x_emb, y_emb, h_emb, w_emb, ln_gamma, ln_beta):
    raise NotImplementedError("write your pallas kernel here")


# speedup vs baseline: 1.1516x; 1.1516x over previous
import jax
import jax.numpy as jnp
from jax import lax
from jax.experimental import pallas as pl
from jax.experimental.pallas import tpu as pltpu
from jax.experimental.pallas import tpu_sc as plsc

VOCAB = 50265
HIDDEN = 768
MAX_POS = 514
MAX_2D = 1024
PAD = 1
EPS = 1e-5
B = 64
S = 512

NC = 2
NS = 16
NW = NC * NS
L = 16
C = 32
ROWS_PER_W = B // NW
NCHUNK = S // C
NG = HIDDEN // L

_GDN = lax.GatherDimensionNumbers(offset_dims=(), collapsed_slice_dims=(0,),
                                  start_index_map=(0,))


def _vgather(v, idx):
    return lax.gather(v, idx[:, None], _GDN, (1,),
                      mode=lax.GatherScatterMode.PROMISE_IN_BOUNDS)


def _lane():
    return jnp.arange(L, dtype=jnp.int32)


def _allsum(v):
    lane = _lane()
    for k in (8, 4, 2, 1):
        v = v + _vgather(v, lane ^ k)
    return v


def _iscan(v):
    lane = _lane()
    zero = jnp.full((L,), 0, jnp.int32)
    for k in (1, 2, 4, 8):
        sh = _vgather(v, jnp.maximum(lane - k, 0))
        v = v + jnp.where(lane >= k, sh, zero)
    return v


def _rsqrt_vec(v):
    i = lax.bitcast_convert_type(v, jnp.int32)
    y = lax.bitcast_convert_type(jnp.int32(0x5F3759DF) - (i >> 1), jnp.float32)
    for _ in range(3):
        y = y * (1.5 - 0.5 * v * y * y)
    return y


def _body(ids_hbm, bbox_hbm, word_hbm, pos_hbm, x_hbm, y_hbm, h_hbm, w_hbm,
          g_hbm, be_hbm, out_hbm,
          ids_v, bbox_v, pidx_v, si0, si1, si2, si3, si4, si5,
          wbuf, pbuf, sbuf, gamma_v, beta_v, sem):
    wid = lax.axis_index("s") * NC + lax.axis_index("c")
    pltpu.sync_copy(g_hbm, gamma_v)
    pltpu.sync_copy(be_hbm, beta_v)

    for r in range(ROWS_PER_W):
        row = wid * ROWS_PER_W + r

        def chunk_body(ci, carry):
            s0 = ci * C
            pltpu.sync_copy(ids_hbm.at[row, pl.ds(s0, C)], ids_v)
            pltpu.sync_copy(bbox_hbm.at[row, pl.ds(s0 * 4, C * 4)], bbox_v)

            ones = jnp.full((L,), 1, jnp.int32)
            zeros = jnp.full((L,), 0, jnp.int32)
            for gi in range(C // L):
                sl = pl.ds(gi * L, L)
                idv = ids_v[sl]
                m = jnp.where(idv != PAD, ones, zeros)
                cs = _iscan(m)
                pidx_v[sl] = (carry + cs) * m + PAD
                carry = carry + _allsum(m)
                lane = _lane()
                v0 = bbox_v[pl.ds(gi * 64 + 0, L)]
                v1 = bbox_v[pl.ds(gi * 64 + 16, L)]
                v2 = bbox_v[pl.ds(gi * 64 + 32, L)]
                v3 = bbox_v[pl.ds(gi * 64 + 48, L)]
                def decol(c):
                    idx = (lane & 3) * 4 + c
                    g0 = _vgather(v0, idx)
                    g1 = _vgather(v1, idx)
                    g2 = _vgather(v2, idx)
                    g3 = _vgather(v3, idx)
                    lo = jnp.where(lane < 4, g0, g1)
                    hi = jnp.where(lane < 12, g2, g3)
                    return jnp.where(lane < 8, lo, hi)
                b0 = decol(0)
                b1 = decol(1)
                b2 = decol(2)
                b3 = decol(3)
                si0[sl] = b0
                si1[sl] = b1
                si2[sl] = b2
                si3[sl] = b3
                si4[sl] = jnp.clip(b3 - b1, 0, MAX_2D - 1)
                si5[sl] = jnp.clip(b2 - b0, 0, MAX_2D - 1)

            descs = [
                pltpu.async_copy(word_hbm.at[ids_v], wbuf, sem),
                pltpu.async_copy(pos_hbm.at[pidx_v], pbuf, sem),
            ]
            spatial = ((x_hbm, si0), (y_hbm, si1), (x_hbm, si2),
                       (y_hbm, si3), (h_hbm, si4), (w_hbm, si5))
            for j, (tbl, idx) in enumerate(spatial):
                descs.append(pltpu.async_copy(tbl.at[idx], sbuf.at[j], sem))
            for d in descs:
                d.wait()

            def tok_body(t, tc):
                acc = jnp.zeros((L,), jnp.float32)
                acc2 = jnp.zeros((L,), jnp.float32)
                for g in range(NG):
                    csl = pl.ds(g * L, L)
                    xv = (wbuf[t, csl] + pbuf[t, csl]
                          + sbuf[g // 8, t, pl.ds((g % 8) * L, L)])
                    acc = acc + xv
                    acc2 = acc2 + xv * xv
                    wbuf[t, csl] = xv
                mean = _allsum(acc) * (1.0 / HIDDEN)
                ex2 = _allsum(acc2) * (1.0 / HIDDEN)
                var = ex2 - mean * mean
                rstd = _rsqrt_vec(var + EPS)
                for g in range(NG):
                    csl = pl.ds(g * L, L)
                    xv = wbuf[t, csl]
                    wbuf[t, csl] = (xv - mean) * rstd * gamma_v[csl] + beta_v[csl]
                return tc

            lax.fori_loop(0, C, tok_body, 0)
            pltpu.sync_copy(wbuf, out_hbm.at[row, pl.ds(s0, C)])
            return carry

        lax.fori_loop(0, NCHUNK, chunk_body, jnp.full((L,), 0, jnp.int32))


def kernel(input_ids, bbox, word_emb, token_type_emb, pos_emb, x_emb, y_emb,
           h_emb, w_emb, ln_gamma, ln_beta):
    pos_comb = pos_emb + token_type_emb[0]

    mesh = plsc.VectorSubcoreMesh(core_axis_name="c", subcore_axis_name="s",
                                  num_cores=NC, num_subcores=NS)
    scratch = [
        pltpu.VMEM((C,), jnp.int32),
        pltpu.VMEM((C * 4,), jnp.int32),
        pltpu.VMEM((C,), jnp.int32),
        pltpu.VMEM((C,), jnp.int32),
        pltpu.VMEM((C,), jnp.int32),
        pltpu.VMEM((C,), jnp.int32),
        pltpu.VMEM((C,), jnp.int32),
        pltpu.VMEM((C,), jnp.int32),
        pltpu.VMEM((C,), jnp.int32),
        pltpu.VMEM((C, HIDDEN), jnp.float32),
        pltpu.VMEM((C, HIDDEN), jnp.float32),
        pltpu.VMEM((6, C, 128), jnp.float32),
        pltpu.VMEM((HIDDEN,), jnp.float32),
        pltpu.VMEM((HIDDEN,), jnp.float32),
        pltpu.SemaphoreType.DMA,
    ]
    f = pl.kernel(
        _body,
        out_type=jax.ShapeDtypeStruct((B, S, HIDDEN), jnp.float32),
        mesh=mesh,
        scratch_types=scratch,
    )
    return f(input_ids, bbox.reshape(B, S * 4), word_emb, pos_comb, x_emb, y_emb, h_emb, w_emb,
             ln_gamma, ln_beta)


# speedup vs baseline: 1.1553x; 1.0032x over previous
import jax
import jax.numpy as jnp
from jax import lax
from jax.experimental import pallas as pl
from jax.experimental.pallas import tpu as pltpu
from jax.experimental.pallas import tpu_sc as plsc

VOCAB = 50265
HIDDEN = 768
MAX_POS = 514
MAX_2D = 1024
PAD = 1
EPS = 1e-5
B = 64
S = 512

NC = 2
NS = 16
NW = NC * NS
L = 16
C = 16
ROWS_PER_W = B // NW
NCHUNK = S // C
NG = HIDDEN // L

_GDN = lax.GatherDimensionNumbers(offset_dims=(), collapsed_slice_dims=(0,),
                                  start_index_map=(0,))


def _vgather(v, idx):
    return lax.gather(v, idx[:, None], _GDN, (1,),
                      mode=lax.GatherScatterMode.PROMISE_IN_BOUNDS)


def _lane():
    return jnp.arange(L, dtype=jnp.int32)


def _allsum(v):
    lane = _lane()
    for k in (8, 4, 2, 1):
        v = v + _vgather(v, lane ^ k)
    return v


def _iscan(v):
    lane = _lane()
    zero = jnp.full((L,), 0, jnp.int32)
    for k in (1, 2, 4, 8):
        sh = _vgather(v, jnp.maximum(lane - k, 0))
        v = v + jnp.where(lane >= k, sh, zero)
    return v


def _rsqrt_vec(v):
    i = lax.bitcast_convert_type(v, jnp.int32)
    y = lax.bitcast_convert_type(jnp.int32(0x5F3759DF) - (i >> 1), jnp.float32)
    for _ in range(3):
        y = y * (1.5 - 0.5 * v * y * y)
    return y


def _body(ids_hbm, bbox_hbm, word_hbm, pos_hbm, spat_hbm,
          g_hbm, be_hbm, out_hbm,
          ids_row, bbox_row,
          idsb0, idsb1, pidx0, pidx1, sidx0, sidx1,
          wraw0, wraw1, pbuf0, pbuf1, sbuf0, sbuf1, obuf0, obuf1,
          gamma_v, beta_v, sem_g0, sem_g1, sem_o):
    wid = lax.axis_index("s") * NC + lax.axis_index("c")
    pltpu.sync_copy(g_hbm, gamma_v)
    pltpu.sync_copy(be_hbm, beta_v)

    idsb = (idsb0, idsb1)
    pidx = (pidx0, pidx1)
    sidx = (sidx0, sidx1)
    wraw = (wraw0, wraw1)
    pbuf = (pbuf0, pbuf1)
    sbuf = (sbuf0, sbuf1)
    obuf = (obuf0, obuf1)
    sem_g = (sem_g0, sem_g1)

    ones = jnp.full((L,), 1, jnp.int32)
    zeros = jnp.full((L,), 0, jnp.int32)
    lane = _lane()

    def index_compute(c, carry, pb):
        idv = ids_row[pl.ds(c * C, L)]
        idsb[pb][...] = idv
        m = jnp.where(idv != PAD, ones, zeros)
        cs = _iscan(m)
        pidx[pb][...] = (carry + cs) * m + PAD
        carry = carry + _allsum(m)

        base = c * (C * 4)
        v0 = bbox_row[pl.ds(base + 0, L)]
        v1 = bbox_row[pl.ds(base + 16, L)]
        v2 = bbox_row[pl.ds(base + 32, L)]
        v3 = bbox_row[pl.ds(base + 48, L)]

        def decol(cc):
            idx = (lane & 3) * 4 + cc
            g0 = _vgather(v0, idx)
            g1 = _vgather(v1, idx)
            g2 = _vgather(v2, idx)
            g3 = _vgather(v3, idx)
            lo = jnp.where(lane < 4, g0, g1)
            hi = jnp.where(lane < 12, g2, g3)
            return jnp.where(lane < 8, lo, hi)

        b0 = decol(0)
        b1 = decol(1)
        b2 = decol(2)
        b3 = decol(3)
        sidx[pb][pl.ds(0 * L, L)] = b0
        sidx[pb][pl.ds(1 * L, L)] = b1 + MAX_2D
        sidx[pb][pl.ds(2 * L, L)] = b2
        sidx[pb][pl.ds(3 * L, L)] = b3 + MAX_2D
        sidx[pb][pl.ds(4 * L, L)] = (jnp.clip(b3 - b1, 0, MAX_2D - 1)
                                     + 2 * MAX_2D)
        sidx[pb][pl.ds(5 * L, L)] = (jnp.clip(b2 - b0, 0, MAX_2D - 1)
                                     + 3 * MAX_2D)
        return carry

    def fire_gathers(pb):
        pltpu.async_copy(word_hbm.at[idsb[pb]], wraw[pb], sem_g[pb])
        pltpu.async_copy(pos_hbm.at[pidx[pb]], pbuf[pb], sem_g[pb])
        pltpu.async_copy(spat_hbm.at[sidx[pb]], sbuf[pb], sem_g[pb])

    def wait_gathers(pb):
        pltpu.make_async_copy(word_hbm.at[idsb[pb]], wraw[pb], sem_g[pb]).wait()
        pltpu.make_async_copy(pos_hbm.at[pidx[pb]], pbuf[pb], sem_g[pb]).wait()
        pltpu.make_async_copy(spat_hbm.at[sidx[pb]], sbuf[pb], sem_g[pb]).wait()

    def compute_chunk(pb):
        wr = wraw[pb]
        pr = pbuf[pb]
        sr = sbuf[pb]
        orf = obuf[pb]

        def tok_body(t, tc):
            acc = jnp.zeros((L,), jnp.float32)
            acc2 = jnp.zeros((L,), jnp.float32)
            for g in range(NG):
                csl = pl.ds(g * L, L)
                xv = (wr[t, csl] + pr[t, csl]
                      + sr[(g // 8) * C + t, pl.ds((g % 8) * L, L)])
                acc = acc + xv
                acc2 = acc2 + xv * xv
                orf[t, csl] = xv
            mean = _allsum(acc) * (1.0 / HIDDEN)
            ex2 = _allsum(acc2) * (1.0 / HIDDEN)
            var = ex2 - mean * mean
            rstd = _rsqrt_vec(var + EPS)
            for g in range(NG):
                csl = pl.ds(g * L, L)
                xv = orf[t, csl]
                orf[t, csl] = (xv - mean) * rstd * gamma_v[csl] + beta_v[csl]
            return tc

        lax.fori_loop(0, C, tok_body, 0)

    for r in range(ROWS_PER_W):
        row = wid * ROWS_PER_W + r
        pltpu.sync_copy(ids_hbm.at[row], ids_row)
        pltpu.sync_copy(bbox_hbm.at[row], bbox_row)

        carry0 = index_compute(0, jnp.full((L,), 0, jnp.int32), 0)
        fire_gathers(0)

        def chunk_step(c, carry, pb, pn):
            carry = index_compute(c + 1, carry, pn)

            @pl.when(c + 1 < NCHUNK)
            def _():
                fire_gathers(pn)

            wait_gathers(pb)

            @pl.when(c >= 2)
            def _():
                pltpu.make_async_copy(obuf[pb], out_hbm.at[row, pl.ds(0, C)],
                                      sem_o).wait()

            compute_chunk(pb)
            pltpu.async_copy(obuf[pb], out_hbm.at[row, pl.ds(c * C, C)], sem_o)
            return carry

        def pair_body(k, carry):
            carry = chunk_step(2 * k, carry, 0, 1)
            carry = chunk_step(2 * k + 1, carry, 1, 0)
            return carry

        lax.fori_loop(0, NCHUNK // 2, pair_body, carry0)

        pltpu.make_async_copy(obuf[0], out_hbm.at[row, pl.ds(0, C)], sem_o).wait()
        pltpu.make_async_copy(obuf[1], out_hbm.at[row, pl.ds(0, C)], sem_o).wait()


def kernel(input_ids, bbox, word_emb, token_type_emb, pos_emb, x_emb, y_emb,
           h_emb, w_emb, ln_gamma, ln_beta):
    pos_comb = pos_emb + token_type_emb[0]
    spat = jnp.concatenate([x_emb, y_emb, h_emb, w_emb], axis=0)

    mesh = plsc.VectorSubcoreMesh(core_axis_name="c", subcore_axis_name="s",
                                  num_cores=NC, num_subcores=NS)
    scratch = [
        pltpu.VMEM((S,), jnp.int32),
        pltpu.VMEM((S * 4,), jnp.int32),
        pltpu.VMEM((C,), jnp.int32),
        pltpu.VMEM((C,), jnp.int32),
        pltpu.VMEM((C,), jnp.int32),
        pltpu.VMEM((C,), jnp.int32),
        pltpu.VMEM((6 * C,), jnp.int32),
        pltpu.VMEM((6 * C,), jnp.int32),
        pltpu.VMEM((C, HIDDEN), jnp.float32),
        pltpu.VMEM((C, HIDDEN), jnp.float32),
        pltpu.VMEM((C, HIDDEN), jnp.float32),
        pltpu.VMEM((C, HIDDEN), jnp.float32),
        pltpu.VMEM((6 * C, 128), jnp.float32),
        pltpu.VMEM((6 * C, 128), jnp.float32),
        pltpu.VMEM((C, HIDDEN), jnp.float32),
        pltpu.VMEM((C, HIDDEN), jnp.float32),
        pltpu.VMEM((HIDDEN,), jnp.float32),
        pltpu.VMEM((HIDDEN,), jnp.float32),
        pltpu.SemaphoreType.DMA,
        pltpu.SemaphoreType.DMA,
        pltpu.SemaphoreType.DMA,
    ]
    f = pl.kernel(
        _body,
        out_type=jax.ShapeDtypeStruct((B, S, HIDDEN), jnp.float32),
        mesh=mesh,
        scratch_types=scratch,
    )
    return f(input_ids, bbox.reshape(B, S * 4), word_emb, pos_comb, spat,
             ln_gamma, ln_beta)
